# baseline (speedup 1.0000x reference)
import functools

import jax
import jax.numpy as jnp
from jax.experimental import pallas as pl
from jax.experimental.pallas import tpu as pltpu

_TOP_K = 8
_BLK = 256
_NSPLIT = 4


def _route_rows(x, w, b1, b2, u, top_k):
    z = jnp.dot(x, w, preferred_element_type=jnp.float32)
    n_experts = z.shape[-1] // 2
    logits = z[:, :n_experts] + b1
    noise_logits = z[:, n_experts:] + b2
    sp = jnp.maximum(noise_logits, 0.0) + jnp.log1p(jnp.exp(-jnp.abs(noise_logits)))
    noisy = logits + u * sp

    col = jax.lax.broadcasted_iota(jnp.int32, noisy.shape, 1).astype(jnp.float32)
    neg_inf = jnp.float32(-jnp.inf)
    big = jnp.float32(n_experts)
    cur = noisy
    tops = []
    idxs = []
    for _ in range(top_k):
        m = jnp.max(cur, axis=1, keepdims=True)
        idx = jnp.min(jnp.where(cur == m, col, big), axis=1, keepdims=True)
        cur = jnp.where(col == idx, neg_inf, cur)
        tops.append(m)
        idxs.append(idx)

    m1 = tops[0]
    denom = sum(jnp.exp(t - m1) for t in tops)
    router = jnp.where(cur < noisy, jnp.exp(noisy - m1), 0.0) / denom
    indices = jnp.concatenate(idxs, axis=1).astype(jnp.int32)
    return router, indices


def _router_block_kernel(*refs, top_k, nsplit):
    xs = refs[:nsplit]
    w1_ref, w2_ref, b1_ref, b2_ref, u_ref = refs[nsplit:nsplit + 5]
    out_ref, idx_ref = refs[nsplit + 5], refs[nsplit + 6]
    w_scratch = refs[nsplit + 7]
    n_experts = w1_ref.shape[1]

    @pl.when(pl.program_id(0) == 0)
    def _pack_weights():
        w_scratch[:, :n_experts] = w1_ref[...]
        w_scratch[:, n_experts:] = w2_ref[...]

    w = w_scratch[...]
    b1 = b1_ref[...]
    b2 = b2_ref[...]
    for j in range(nsplit):
        router, indices = _route_rows(xs[j][0], w, b1, b2, u_ref[j], top_k)
        out_ref[j] = router
        idx_ref[j] = indices


_NOISE_CACHE = {}


def _noise_table(n_tokens, n_experts, nsplit):
    k = (n_tokens, n_experts, nsplit)
    v = _NOISE_CACHE.get(k)
    if v is None:
        def build():
            u = jax.random.uniform(
                jax.random.key(42), (n_tokens, n_experts), dtype=jnp.float32
            )
            return u.reshape(nsplit, n_tokens // nsplit, n_experts)
        try:
            with jax.ensure_compile_time_eval():
                v = build()
            _NOISE_CACHE[k] = v
        except Exception:
            v = build()
    return v


def kernel(x, W1, b1, W2, b2):
    n_tokens, n_embed = x.shape
    n_experts = W1.shape[1]
    ns = _NSPLIT
    q = n_tokens // ns
    blk = min(_BLK, q)
    grid = (q // blk,)
    xq = x.reshape(ns, q, n_embed)
    uq = _noise_table(n_tokens, n_experts, ns)
    xspec = lambda j: pl.BlockSpec((1, blk, n_embed), lambda i, j=j: (j, i, 0))
    router, indices = pl.pallas_call(
        functools.partial(_router_block_kernel, top_k=_TOP_K, nsplit=ns),
        grid=grid,
        in_specs=[xspec(j) for j in range(ns)] + [
            pl.BlockSpec((n_embed, n_experts), lambda i: (0, 0)),
            pl.BlockSpec((n_embed, n_experts), lambda i: (0, 0)),
            pl.BlockSpec((1, n_experts), lambda i: (0, 0)),
            pl.BlockSpec((1, n_experts), lambda i: (0, 0)),
            pl.BlockSpec((ns, blk, n_experts), lambda i: (0, i, 0)),
        ],
        out_specs=[
            pl.BlockSpec((ns, blk, n_experts), lambda i: (0, i, 0)),
            pl.BlockSpec((ns, blk, _TOP_K), lambda i: (0, i, 0)),
        ],
        out_shape=[
            jax.ShapeDtypeStruct((ns, q, n_experts), jnp.float32),
            jax.ShapeDtypeStruct((ns, q, _TOP_K), jnp.int32),
        ],
        scratch_shapes=[pltpu.VMEM((n_embed, 2 * n_experts), jnp.float32)],
        compiler_params=pltpu.CompilerParams(
            dimension_semantics=("arbitrary",),
        ),
    )(*([xq] * ns), W1, W2, b1[None, :], b2[None, :], uq)
    return (router.reshape(n_tokens, n_experts),
            indices.reshape(n_tokens, _TOP_K))


# speedup vs baseline: 1.0379x; 1.0379x over previous
import functools

import jax
import jax.numpy as jnp
from jax.experimental import pallas as pl
from jax.experimental.pallas import tpu as pltpu

_TOP_K = 8
_BLK = 256
_NSPLIT = 4


def _route_rows(x, w, b1, b2, u, top_k):
    z = jnp.dot(x, w, preferred_element_type=jnp.float32)
    n_experts = z.shape[-1] // 2
    logits = z[:, :n_experts] + b1
    noise_logits = z[:, n_experts:] + b2
    sp = jnp.maximum(noise_logits, 0.0) + jnp.log1p(jnp.exp(-jnp.abs(noise_logits)))
    noisy = logits + u * sp

    col = jax.lax.broadcasted_iota(jnp.int32, noisy.shape, 1).astype(jnp.float32)
    neg_inf = jnp.float32(-jnp.inf)
    big = jnp.float32(n_experts)
    cur = noisy
    tops = []
    idxs = []
    for _ in range(top_k):
        m = jnp.max(cur, axis=1, keepdims=True)
        idx = jnp.min(jnp.where(cur == m, col, big), axis=1, keepdims=True)
        cur = jnp.where(col == idx, neg_inf, cur)
        tops.append(m)
        idxs.append(idx)

    m1 = tops[0]
    denom = sum(jnp.exp(t - m1) for t in tops)
    router = jnp.where(cur < noisy, jnp.exp(noisy - m1), 0.0) / denom
    indices = jnp.concatenate(idxs, axis=1).astype(jnp.int32)
    return router, indices


def _router_block_kernel(*refs, top_k, nsplit, blk):
    xs = refs[:nsplit]
    w1_ref, w2_ref, b1_ref, b2_ref, u_ref = refs[nsplit:nsplit + 5]
    out_ref, idx_ref = refs[nsplit + 5], refs[nsplit + 6]
    w_scratch = refs[nsplit + 7]
    n_experts = w1_ref.shape[1]

    @pl.when(pl.program_id(0) == 0)
    def _pack_weights():
        w_scratch[:, :n_experts] = w1_ref[...]
        w_scratch[:, n_experts:] = w2_ref[...]

    w = w_scratch[...]
    b1 = b1_ref[...]
    b2 = b2_ref[...]
    for j in range(nsplit):
        lo, hi = j * blk, (j + 1) * blk
        router, indices = _route_rows(xs[j][0], w, b1, b2, u_ref[lo:hi], top_k)
        out_ref[lo:hi] = router
        idx_ref[lo:hi] = indices


_NOISE_CACHE = {}


def _noise_table(n_tokens, n_experts):
    k = (n_tokens, n_experts)
    v = _NOISE_CACHE.get(k)
    if v is None:
        def build():
            return jax.random.uniform(
                jax.random.key(42), (n_tokens, n_experts), dtype=jnp.float32
            )
        try:
            with jax.ensure_compile_time_eval():
                v = build()
            _NOISE_CACHE[k] = v
        except Exception:
            v = build()
    return v


def kernel(x, W1, b1, W2, b2):
    n_tokens, n_embed = x.shape
    n_experts = W1.shape[1]
    u = _noise_table(n_tokens, n_experts)

    ns = _NSPLIT
    blk = min(_BLK, n_tokens // ns)
    nblocks = n_tokens // blk
    grid = (nblocks // ns,)
    xb = x.reshape(nblocks, blk, n_embed)
    xspec = lambda j: pl.BlockSpec(
        (1, blk, n_embed), lambda i, j=j: (ns * i + j, 0, 0))
    router, indices = pl.pallas_call(
        functools.partial(_router_block_kernel, top_k=_TOP_K, nsplit=ns,
                          blk=blk),
        grid=grid,
        in_specs=[xspec(j) for j in range(ns)] + [
            pl.BlockSpec((n_embed, n_experts), lambda i: (0, 0)),
            pl.BlockSpec((n_embed, n_experts), lambda i: (0, 0)),
            pl.BlockSpec((1, n_experts), lambda i: (0, 0)),
            pl.BlockSpec((1, n_experts), lambda i: (0, 0)),
            pl.BlockSpec((ns * blk, n_experts), lambda i: (i, 0)),
        ],
        out_specs=[
            pl.BlockSpec((ns * blk, n_experts), lambda i: (i, 0)),
            pl.BlockSpec((ns * blk, _TOP_K), lambda i: (i, 0)),
        ],
        out_shape=[
            jax.ShapeDtypeStruct((n_tokens, n_experts), jnp.float32),
            jax.ShapeDtypeStruct((n_tokens, _TOP_K), jnp.int32),
        ],
        scratch_shapes=[pltpu.VMEM((n_embed, 2 * n_experts), jnp.float32)],
        compiler_params=pltpu.CompilerParams(
            dimension_semantics=("arbitrary",),
        ),
    )(*([xb] * ns), W1, W2, b1[None, :], b2[None, :], u)
    return router, indices
